# baseline (speedup 1.0000x reference)
import jax
import jax.numpy as jnp
from jax.experimental import pallas as pl
from jax.experimental.pallas import tpu as pltpu
from jax.experimental.pallas import tpu_sc as plsc

_D = 256
_T = 8192
_C = 8192
_BM = 512
_TB = _T // _BM
_NCHUNK = _C // 128
_COMMIT = 0.25


def _e2_body(e_ref, e2_ref, et_ref):
    e = e_ref[...]
    e2_ref[...] = jnp.sum(e * e, axis=0)[None, :]
    et_ref[...] = e.T


def _e2_and_transpose(embed):
    return pl.pallas_call(
        _e2_body,
        grid=(8,),
        in_specs=[pl.BlockSpec((_D, _C // 8), lambda j: (0, j))],
        out_specs=[
            pl.BlockSpec((1, _C // 8), lambda j: (0, j)),
            pl.BlockSpec((_C // 8, _D), lambda j: (j, 0)),
        ],
        out_shape=[
            jax.ShapeDtypeStruct((1, _C), jnp.float32),
            jax.ShapeDtypeStruct((_C, _D), jnp.float32),
        ],
    )(embed)


_BJ = 1024


def _argmin_body(x_ref, e_ref, e2_ref, idx_ref, loss_ref, acc_ref):
    t = pl.program_id(0)
    e = e_ref[...]
    x = x_ref[...]
    x2x = (x + x).astype(jnp.bfloat16)

    lane = jax.lax.broadcasted_iota(jnp.int32, (128, 128), 1)
    lsum = 0.0
    for r in range(_BM // 128):
        xr = x2x[r * 128:(r + 1) * 128, :]
        rmin = jnp.full((128, 128), jnp.inf, jnp.float32)
        rarg = jnp.zeros((128, 128), jnp.int32)
        for j in range(_C // _BJ):
            mmj = jnp.dot(xr, e[:, j * _BJ:(j + 1) * _BJ],
                          preferred_element_type=jnp.float32,
                          precision=jax.lax.Precision.DEFAULT)
            for kk in range(_BJ // 128):
                k = j * (_BJ // 128) + kk
                e2c = e2_ref[:, k * 128:(k + 1) * 128]
                sc = e2c - mmj[:, kk * 128:(kk + 1) * 128]
                upd = sc < rmin
                rmin = jnp.minimum(sc, rmin)
                rarg = jnp.where(upd, k, rarg)
        full = rarg * 128 + lane
        gmin = jnp.min(rmin, axis=1, keepdims=True)
        cand = jnp.where(rmin == gmin, full, _C)
        idx_ref[pl.ds(r * 128, 128)] = jnp.min(cand, axis=1)
        lsum += jnp.sum(gmin)

    @pl.when(t == 0)
    def _():
        acc_ref[0] = 0.0

    acc_ref[0] += jnp.sum(x * x) + lsum

    @pl.when(t == pl.num_programs(0) - 1)
    def _():
        loss_ref[0] = acc_ref[0]


def _argmin_loss(x, embed, e2, t0, nblk):
    return pl.pallas_call(
        _argmin_body,
        grid=(nblk,),
        in_specs=[
            pl.BlockSpec((_BM, _D), lambda t: (t0 + t, 0)),
            pl.BlockSpec((_D, _C), lambda t: (0, 0)),
            pl.BlockSpec((1, _C), lambda t: (0, 0)),
        ],
        out_specs=[
            pl.BlockSpec((_BM,), lambda t: (t,)),
            pl.BlockSpec(memory_space=pltpu.SMEM),
        ],
        out_shape=[
            jax.ShapeDtypeStruct((nblk * _BM,), jnp.int32),
            jax.ShapeDtypeStruct((1,), jnp.float32),
        ],
        scratch_shapes=[
            pltpu.SMEM((1,), jnp.float32),
        ],
    )(x, embed.astype(jnp.bfloat16), e2)


_GW = 128


def _gather_codes(embed_t, indices):
    n = indices.shape[0]
    idx2 = indices.reshape(1, n)

    @pl.kernel(
        out_type=jax.ShapeDtypeStruct((n, _D), jnp.float32),
        mesh=plsc.VectorSubcoreMesh(core_axis_name="core",
                                    subcore_axis_name="subcore"),
    )
    def k(x_hbm, i_hbm, o_hbm):
        def body(i_vmem, o_vmem):
            pltpu.sync_copy(x_hbm.at[i_vmem.at[0]], o_vmem)

        pltpu.emit_pipeline(
            body,
            grid=(n // _GW,),
            in_specs=[pl.BlockSpec((1, _GW), index_map=lambda i: (0, i))],
            out_specs=[pl.BlockSpec((_GW, _D), index_map=lambda i: (i, 0))],
            core_axis_name=("core", "subcore"),
            dimension_semantics=(pltpu.PARALLEL,),
        )(i_hbm, o_hbm)

    return k(embed_t, idx2)


def kernel(inputs, embed):
    x = inputs.reshape(_T, _D)
    e2, embed_t = _e2_and_transpose(embed)
    hb = _TB // 2
    idx1, s1 = _argmin_loss(x, embed, e2, 0, hb)
    idx2, s2 = _argmin_loss(x, embed, e2, hb, hb)
    q1 = _gather_codes(embed_t, idx1)
    q2 = _gather_codes(embed_t, idx2)
    quantized = jnp.concatenate([q1, q2]).reshape(inputs.shape)
    loss = (s1[0] + s2[0]) * (_COMMIT / (_T * _D))
    indices = jnp.concatenate([idx1, idx2])
    return quantized, loss, indices


# speedup vs baseline: 1.0490x; 1.0490x over previous
import jax
import jax.numpy as jnp
from jax.experimental import pallas as pl
from jax.experimental.pallas import tpu as pltpu
from jax.experimental.pallas import tpu_sc as plsc

_D = 256
_T = 8192
_C = 8192
_BM = 512
_TB = _T // _BM
_NCHUNK = _C // 128
_COMMIT = 0.25


def _e2_body(e_ref, e2_ref, et_ref, ebf_ref):
    e = e_ref[...]
    e2_ref[...] = jnp.sum(e * e, axis=0)[None, :]
    et_ref[...] = e.T
    ebf_ref[...] = e.astype(jnp.bfloat16)


def _e2_and_transpose(embed):
    return pl.pallas_call(
        _e2_body,
        grid=(8,),
        in_specs=[pl.BlockSpec((_D, _C // 8), lambda j: (0, j))],
        out_specs=[
            pl.BlockSpec((1, _C // 8), lambda j: (0, j)),
            pl.BlockSpec((_C // 8, _D), lambda j: (j, 0)),
            pl.BlockSpec((_D, _C // 8), lambda j: (0, j)),
        ],
        out_shape=[
            jax.ShapeDtypeStruct((1, _C), jnp.float32),
            jax.ShapeDtypeStruct((_C, _D), jnp.float32),
            jax.ShapeDtypeStruct((_D, _C), jnp.bfloat16),
        ],
    )(embed)


_BJ = 1024


def _argmin_body(x_ref, e_ref, e2_ref, idx_ref, loss_ref, acc_ref):
    t = pl.program_id(0)
    e = e_ref[...]
    x = x_ref[...]
    x2x = (x + x).astype(jnp.bfloat16)

    lane = jax.lax.broadcasted_iota(jnp.int32, (128, 128), 1)
    lsum = 0.0
    for r in range(_BM // 128):
        xr = x2x[r * 128:(r + 1) * 128, :]
        rmin = jnp.full((128, 128), jnp.inf, jnp.float32)
        rarg = jnp.zeros((128, 128), jnp.int32)
        for j in range(_C // _BJ):
            mmj = jnp.dot(xr, e[:, j * _BJ:(j + 1) * _BJ],
                          preferred_element_type=jnp.float32,
                          precision=jax.lax.Precision.DEFAULT)
            for kk in range(_BJ // 128):
                k = j * (_BJ // 128) + kk
                e2c = e2_ref[:, k * 128:(k + 1) * 128]
                sc = e2c - mmj[:, kk * 128:(kk + 1) * 128]
                upd = sc < rmin
                rmin = jnp.minimum(sc, rmin)
                rarg = jnp.where(upd, k, rarg)
        full = rarg * 128 + lane
        gmin = jnp.min(rmin, axis=1, keepdims=True)
        cand = jnp.where(rmin == gmin, full, _C)
        idx_ref[pl.ds(r * 128, 128)] = jnp.min(cand, axis=1)
        lsum += jnp.sum(gmin)

    @pl.when(t == 0)
    def _():
        acc_ref[0] = 0.0

    acc_ref[0] += jnp.sum(x * x) + lsum

    @pl.when(t == pl.num_programs(0) - 1)
    def _():
        loss_ref[0] = acc_ref[0]


def _argmin_loss(x, embed_bf, e2, t0, nblk):
    return pl.pallas_call(
        _argmin_body,
        grid=(nblk,),
        in_specs=[
            pl.BlockSpec((_BM, _D), lambda t: (t0 + t, 0)),
            pl.BlockSpec((_D, _C), lambda t: (0, 0)),
            pl.BlockSpec((1, _C), lambda t: (0, 0)),
        ],
        out_specs=[
            pl.BlockSpec((_BM,), lambda t: (t,)),
            pl.BlockSpec(memory_space=pltpu.SMEM),
        ],
        out_shape=[
            jax.ShapeDtypeStruct((nblk * _BM,), jnp.int32),
            jax.ShapeDtypeStruct((1,), jnp.float32),
        ],
        scratch_shapes=[
            pltpu.SMEM((1,), jnp.float32),
        ],
    )(x, embed_bf, e2)


_GW = 128


def _gather_codes(embed_t, indices):
    n = indices.shape[0]
    idx2 = indices.reshape(1, n)

    @pl.kernel(
        out_type=jax.ShapeDtypeStruct((n, _D), jnp.float32),
        mesh=plsc.VectorSubcoreMesh(core_axis_name="core",
                                    subcore_axis_name="subcore"),
    )
    def k(x_hbm, i_hbm, o_hbm):
        def body(i_vmem, o_vmem):
            pltpu.sync_copy(x_hbm.at[i_vmem.at[0]], o_vmem)

        pltpu.emit_pipeline(
            body,
            grid=(n // _GW,),
            in_specs=[pl.BlockSpec((1, _GW), index_map=lambda i: (0, i))],
            out_specs=[pl.BlockSpec((_GW, _D), index_map=lambda i: (i, 0))],
            core_axis_name=("core", "subcore"),
            dimension_semantics=(pltpu.PARALLEL,),
        )(i_hbm, o_hbm)

    return k(embed_t, idx2)


def kernel(inputs, embed):
    x = inputs.reshape(_T, _D)
    e2, embed_t, embed_bf = _e2_and_transpose(embed)
    hb = _TB // 2
    idx1, s1 = _argmin_loss(x, embed_bf, e2, 0, hb)
    idx2, s2 = _argmin_loss(x, embed_bf, e2, hb, hb)
    q1 = _gather_codes(embed_t, idx1)
    q2 = _gather_codes(embed_t, idx2)
    quantized = jnp.concatenate([q1, q2]).reshape(inputs.shape)
    loss = (s1[0] + s2[0]) * (_COMMIT / (_T * _D))
    indices = jnp.concatenate([idx1, idx2])
    return quantized, loss, indices


# speedup vs baseline: 1.0683x; 1.0184x over previous
import jax
import jax.numpy as jnp
from jax.experimental import pallas as pl
from jax.experimental.pallas import tpu as pltpu
from jax.experimental.pallas import tpu_sc as plsc

_D = 256
_T = 8192
_C = 8192
_BM = 512
_TB = _T // _BM
_NCHUNK = _C // 128
_COMMIT = 0.25


def _e2_body(e_ref, e2_ref, et_ref, ebf_ref):
    e = e_ref[...]
    e2_ref[...] = jnp.sum(e * e, axis=0)[None, :]
    et_ref[...] = e.T
    ebf_ref[...] = e.astype(jnp.bfloat16)


def _e2_and_transpose(embed):
    return pl.pallas_call(
        _e2_body,
        grid=(8,),
        in_specs=[pl.BlockSpec((_D, _C // 8), lambda j: (0, j))],
        out_specs=[
            pl.BlockSpec((1, _C // 8), lambda j: (0, j)),
            pl.BlockSpec((_C // 8, _D), lambda j: (j, 0)),
            pl.BlockSpec((_D, _C // 8), lambda j: (0, j)),
        ],
        out_shape=[
            jax.ShapeDtypeStruct((1, _C), jnp.float32),
            jax.ShapeDtypeStruct((_C, _D), jnp.float32),
            jax.ShapeDtypeStruct((_D, _C), jnp.bfloat16),
        ],
    )(embed)


_BJ = 1024


def _argmin_body(x_ref, e_ref, e2_ref, idx_ref, loss_ref, acc_ref):
    t = pl.program_id(0)
    e = e_ref[...]
    x = x_ref[...]
    x2x = (x + x).astype(jnp.bfloat16)

    lane = jax.lax.broadcasted_iota(jnp.int32, (128, 128), 1)
    lsum = 0.0
    for r in range(_BM // 128):
        xr = x2x[r * 128:(r + 1) * 128, :]
        rmin = jnp.full((128, 128), jnp.inf, jnp.float32)
        rarg = jnp.zeros((128, 128), jnp.int32)
        for j in range(_C // _BJ):
            mmj = jnp.dot(xr, e[:, j * _BJ:(j + 1) * _BJ],
                          preferred_element_type=jnp.float32,
                          precision=jax.lax.Precision.DEFAULT)
            for kk in range(_BJ // 128):
                k = j * (_BJ // 128) + kk
                e2c = e2_ref[:, k * 128:(k + 1) * 128]
                sc = e2c - mmj[:, kk * 128:(kk + 1) * 128]
                upd = sc < rmin
                rmin = jnp.minimum(sc, rmin)
                rarg = jnp.where(upd, k, rarg)
        full = rarg * 128 + lane
        gmin = jnp.min(rmin, axis=1, keepdims=True)
        cand = jnp.where(rmin == gmin, full, _C)
        idx_ref[pl.ds(r * 128, 128)] = jnp.min(cand, axis=1)
        lsum += jnp.sum(gmin)

    @pl.when(t == 0)
    def _():
        acc_ref[0] = 0.0

    acc_ref[0] += jnp.sum(x * x) + lsum

    @pl.when(t == pl.num_programs(0) - 1)
    def _():
        loss_ref[0] = acc_ref[0]


def _argmin_loss(x, embed_bf, e2, t0, nblk):
    return pl.pallas_call(
        _argmin_body,
        grid=(nblk,),
        in_specs=[
            pl.BlockSpec((_BM, _D), lambda t: (t0 + t, 0)),
            pl.BlockSpec((_D, _C), lambda t: (0, 0)),
            pl.BlockSpec((1, _C), lambda t: (0, 0)),
        ],
        out_specs=[
            pl.BlockSpec((_BM,), lambda t: (t,)),
            pl.BlockSpec(memory_space=pltpu.SMEM),
        ],
        out_shape=[
            jax.ShapeDtypeStruct((nblk * _BM,), jnp.int32),
            jax.ShapeDtypeStruct((1,), jnp.float32),
        ],
        scratch_shapes=[
            pltpu.SMEM((1,), jnp.float32),
        ],
    )(x, embed_bf, e2)


_GW = 128


def _gather_codes(embed_t, indices):
    n = indices.shape[0]
    idx2 = indices.reshape(1, n)

    @pl.kernel(
        out_type=jax.ShapeDtypeStruct((n, _D), jnp.float32),
        mesh=plsc.VectorSubcoreMesh(core_axis_name="core",
                                    subcore_axis_name="subcore"),
    )
    def k(x_hbm, i_hbm, o_hbm):
        def body(i_vmem, o_vmem):
            pltpu.sync_copy(x_hbm.at[i_vmem.at[0]], o_vmem)

        pltpu.emit_pipeline(
            body,
            grid=(n // _GW,),
            in_specs=[pl.BlockSpec((1, _GW), index_map=lambda i: (0, i))],
            out_specs=[pl.BlockSpec((_GW, _D), index_map=lambda i: (i, 0))],
            core_axis_name=("core", "subcore"),
            dimension_semantics=(pltpu.PARALLEL,),
        )(i_hbm, o_hbm)

    return k(embed_t, idx2)


def kernel(inputs, embed):
    x = inputs.reshape(_T, _D)
    e2, embed_t, embed_bf = _e2_and_transpose(embed)
    indices, s = _argmin_loss(x, embed_bf, e2, 0, _TB)
    quantized = _gather_codes(embed_t, indices).reshape(inputs.shape)
    loss = s[0] * (_COMMIT / (_T * _D))
    return quantized, loss, indices


# speedup vs baseline: 1.1747x; 1.0996x over previous
import jax
import jax.numpy as jnp
from jax.experimental import pallas as pl
from jax.experimental.pallas import tpu as pltpu
from jax.experimental.pallas import tpu_sc as plsc

_D = 256
_T = 8192
_C = 8192
_BM = 512
_TB = _T // _BM
_NCHUNK = _C // 128
_COMMIT = 0.25


_BJ = 1024


def _argmin_body(x_ref, ef_ref, idx_ref, loss_ref, et_ref,
                 ebf_s, e2_s, acc_ref):
    t = pl.program_id(0)

    @pl.when(t == 0)
    def _():
        ef = ef_ref[...]
        ebf_s[...] = ef.astype(jnp.bfloat16)
        e2_s[...] = jnp.sum(ef * ef, axis=0)[None, :]
        acc_ref[0] = 0.0

    et_ref[...] = ef_ref[:, pl.ds(t * _BM, _BM)].T

    e = ebf_s[...]
    x = x_ref[...]
    x2x = (x + x).astype(jnp.bfloat16)

    lane = jax.lax.broadcasted_iota(jnp.int32, (128, 128), 1)
    lsum = 0.0
    for r in range(_BM // 128):
        xr = x2x[r * 128:(r + 1) * 128, :]
        rmin = jnp.full((128, 128), jnp.inf, jnp.float32)
        rarg = jnp.zeros((128, 128), jnp.int32)
        for j in range(_C // _BJ):
            mmj = jnp.dot(xr, e[:, j * _BJ:(j + 1) * _BJ],
                          preferred_element_type=jnp.float32,
                          precision=jax.lax.Precision.DEFAULT)
            for kk in range(_BJ // 128):
                k = j * (_BJ // 128) + kk
                e2c = e2_s[:, k * 128:(k + 1) * 128]
                sc = e2c - mmj[:, kk * 128:(kk + 1) * 128]
                upd = sc < rmin
                rmin = jnp.minimum(sc, rmin)
                rarg = jnp.where(upd, k, rarg)
        full = rarg * 128 + lane
        gmin = jnp.min(rmin, axis=1, keepdims=True)
        cand = jnp.where(rmin == gmin, full, _C)
        idx_ref[pl.ds(r * 128, 128)] = jnp.min(cand, axis=1)
        lsum += jnp.sum(gmin)

    acc_ref[0] += jnp.sum(x * x) + lsum

    @pl.when(t == pl.num_programs(0) - 1)
    def _():
        loss_ref[0] = acc_ref[0]


def _argmin_loss(x, embed):
    return pl.pallas_call(
        _argmin_body,
        grid=(_TB,),
        in_specs=[
            pl.BlockSpec((_BM, _D), lambda t: (t, 0)),
            pl.BlockSpec((_D, _C), lambda t: (0, 0)),
        ],
        out_specs=[
            pl.BlockSpec((_BM,), lambda t: (t,)),
            pl.BlockSpec(memory_space=pltpu.SMEM),
            pl.BlockSpec((_BM, _D), lambda t: (t, 0)),
        ],
        out_shape=[
            jax.ShapeDtypeStruct((_T,), jnp.int32),
            jax.ShapeDtypeStruct((1,), jnp.float32),
            jax.ShapeDtypeStruct((_C, _D), jnp.float32),
        ],
        scratch_shapes=[
            pltpu.VMEM((_D, _C), jnp.bfloat16),
            pltpu.VMEM((1, _C), jnp.float32),
            pltpu.SMEM((1,), jnp.float32),
        ],
    )(x, embed)


_GW = 128


def _gather_codes(embed_t, indices):
    n = indices.shape[0]
    idx2 = indices.reshape(1, n)

    @pl.kernel(
        out_type=jax.ShapeDtypeStruct((n, _D), jnp.float32),
        mesh=plsc.VectorSubcoreMesh(core_axis_name="core",
                                    subcore_axis_name="subcore"),
    )
    def k(x_hbm, i_hbm, o_hbm):
        def body(i_vmem, o_vmem):
            pltpu.sync_copy(x_hbm.at[i_vmem.at[0]], o_vmem)

        pltpu.emit_pipeline(
            body,
            grid=(n // _GW,),
            in_specs=[pl.BlockSpec((1, _GW), index_map=lambda i: (0, i))],
            out_specs=[pl.BlockSpec((_GW, _D), index_map=lambda i: (i, 0))],
            core_axis_name=("core", "subcore"),
            dimension_semantics=(pltpu.PARALLEL,),
        )(i_hbm, o_hbm)

    return k(embed_t, idx2)


def kernel(inputs, embed):
    x = inputs.reshape(_T, _D)
    indices, s, embed_t = _argmin_loss(x, embed)
    quantized = _gather_codes(embed_t, indices).reshape(inputs.shape)
    loss = s[0] * (_COMMIT / (_T * _D))
    return quantized, loss, indices
